# baseline (speedup 1.0000x reference)
import jax
import jax.numpy as jnp
from jax import lax
from jax.experimental import pallas as pl
from jax.experimental.pallas import tpu as pltpu
from jax.experimental.pallas import tpu_sc as plsc

L = 16
NC = 2
NS = 16
NW = NC * NS
CH = 128
NBLK = 3

_SC_PARAMS = dict(
    compiler_params=pltpu.CompilerParams(needs_layout_passes=False),
)


def _make_deg_kernel(npad, nblk, bs):
    rpt = npad // NS

    def body(col3d, ew3d, degp_hbm, idx_c, ewb, zb, deg_sh, sem):
        c = lax.axis_index("c")
        s = lax.axis_index("s")
        wid = s * NC + c
        base_row = s * rpt

        zeros16 = jnp.zeros((L,), jnp.float32)
        for q in range(CH // L):
            zb[pl.ds(q * L, L)] = zeros16

        @pl.loop(0, rpt // CH)
        def _zd(k):
            pltpu.sync_copy(zb, deg_sh.at[pl.ds(base_row + k * CH, CH)])

        plsc.subcore_barrier()

        for b in range(nblk):
            pltpu.sync_copy(col3d.at[wid, b], idx_c)
            pltpu.sync_copy(ew3d.at[wid, b], ewb)

            @pl.loop(0, bs)
            def _fire(j):
                pltpu.async_copy(ewb.at[j], deg_sh.at[idx_c.at[j]], sem,
                                 add=True)

            @pl.loop(0, bs)
            def _drain(j):
                pltpu.make_async_copy(ewb.at[j], deg_sh.at[idx_c.at[j]],
                                      sem).wait()

        plsc.subcore_barrier()
        pltpu.sync_copy(deg_sh.at[pl.ds(base_row, rpt)],
                        degp_hbm.at[c, pl.ds(base_row, rpt)])

    mesh = plsc.VectorSubcoreMesh(core_axis_name="c", subcore_axis_name="s")
    return pl.kernel(
        body,
        out_type=jax.ShapeDtypeStruct((NC, npad), jnp.float32),
        mesh=mesh,
        scratch_types=[
            pltpu.VMEM((bs, CH), jnp.int32),
            pltpu.VMEM((bs, CH), jnp.float32),
            pltpu.VMEM((CH,), jnp.float32),
            pltpu.VMEM_SHARED((npad,), jnp.float32),
            pltpu.SemaphoreType.DMA,
        ],
        **_SC_PARAMS,
    )


def _make_edge_kernel(npad, d_out, nblk, bs):
    rpt = npad // NS
    qn = d_out // L

    def body(row3d, col3d, ew3d, h2, out_hbm,
             idx_r, idx_c, ewb, rows_a, rows_b, zbuf, acc_sh, sga, sgb):
        c = lax.axis_index("c")
        s = lax.axis_index("s")
        wid = s * NC + c
        base_row = s * rpt

        zeros16 = jnp.zeros((L,), jnp.float32)

        @pl.loop(0, 8)
        def _z(i):
            for q in range(qn):
                zbuf[i, pl.ds(q * L, L)] = zeros16

        @pl.loop(0, rpt // 8)
        def _za(k):
            pltpu.sync_copy(zbuf, acc_sh.at[pl.ds(base_row + k * 8, 8)])

        plsc.subcore_barrier()

        def proc(j, buf, sem):
            pltpu.async_copy(h2.at[idx_r.at[j]], buf, sem).wait()
            for g in range(CH // L):
                ev = ewb[j, pl.ds(g * L, L)]
                for i in range(L):
                    w = ev[i]
                    e_idx = g * L + i
                    for q in range(qn):
                        buf[e_idx, pl.ds(q * L, L)] = (
                            buf[e_idx, pl.ds(q * L, L)] * w)
            pltpu.sync_copy(buf, acc_sh.at[idx_c.at[j]], add=True)

        for b in range(nblk):
            pltpu.sync_copy(row3d.at[wid, b], idx_r)
            pltpu.sync_copy(col3d.at[wid, b], idx_c)
            pltpu.sync_copy(ew3d.at[wid, b], ewb)

            @pl.loop(0, bs // 2)
            def _pair(it):
                j0 = 2 * it
                proc(j0, rows_a, sga)
                proc(j0 + 1, rows_b, sgb)

        plsc.subcore_barrier()
        pltpu.sync_copy(acc_sh.at[pl.ds(base_row, rpt)],
                        out_hbm.at[c, pl.ds(base_row, rpt)])

    mesh = plsc.VectorSubcoreMesh(core_axis_name="c", subcore_axis_name="s")
    return pl.kernel(
        body,
        out_type=jax.ShapeDtypeStruct((NC, npad, d_out), jnp.float32),
        mesh=mesh,
        scratch_types=[
            pltpu.VMEM((bs, CH), jnp.int32),
            pltpu.VMEM((bs, CH), jnp.int32),
            pltpu.VMEM((bs, CH), jnp.float32),
            pltpu.VMEM((CH, d_out), jnp.float32),
            pltpu.VMEM((CH, d_out), jnp.float32),
            pltpu.VMEM((8, d_out), jnp.float32),
            pltpu.VMEM_SHARED((npad, d_out), jnp.float32),
            pltpu.SemaphoreType.DMA,
            pltpu.SemaphoreType.DMA,
        ],
        **_SC_PARAMS,
    )


def _dis_block(degp_blk):
    deg = degp_blk[0] + degp_blk[1]
    return jnp.where(deg > 0.0, lax.rsqrt(jnp.where(deg > 0.0, deg, 1.0)),
                     0.0)


def _matmul_body(x_ref, w_ref, degp_ref, o_ref):
    dis = _dis_block(degp_ref[...])
    o_ref[...] = jnp.dot(x_ref[...], w_ref[...],
                         preferred_element_type=jnp.float32) * dis[:, None]


def _combine_body(p_ref, degp_ref, o_ref):
    dis = _dis_block(degp_ref[...])
    o_ref[...] = (p_ref[0] + p_ref[1]) * dis[:, None]


def kernel(x, edge_index, edge_weight, W):
    n, d_in = x.shape
    d_out = W.shape[1]
    e = edge_weight.shape[0]

    loop_idx = jnp.arange(n, dtype=edge_index.dtype)
    row = jnp.concatenate([edge_index[0], loop_idx])
    col = jnp.concatenate([edge_index[1], loop_idx])
    ew = jnp.concatenate([edge_weight, jnp.ones((n,), edge_weight.dtype)])
    e_tot = e + n
    grp = NW * CH
    cpw = (e_tot + grp - 1) // grp
    cpw = ((cpw + 2 * NBLK - 1) // (2 * NBLK)) * (2 * NBLK)
    e_pad = cpw * grp
    pad = e_pad - e_tot
    bs = cpw // NBLK
    shp = (NW, NBLK, bs, CH)
    row = jnp.concatenate([row, jnp.zeros((pad,), row.dtype)]).reshape(shp)
    col = jnp.concatenate([col, jnp.zeros((pad,), col.dtype)]).reshape(shp)
    ew = jnp.concatenate([ew, jnp.zeros((pad,), ew.dtype)]).reshape(shp)

    rpt = ((n + NS * CH - 1) // (NS * CH)) * CH
    npad = rpt * NS

    degp = _make_deg_kernel(npad, NBLK, bs)(col, ew)

    xp = jnp.concatenate(
        [x, jnp.zeros((npad - n, d_in), x.dtype)]) if npad > n else x
    bm = 1024
    h2 = pl.pallas_call(
        _matmul_body,
        grid=(npad // bm,),
        in_specs=[pl.BlockSpec((bm, d_in), lambda i: (i, 0)),
                  pl.BlockSpec((d_in, d_out), lambda i: (0, 0)),
                  pl.BlockSpec((NC, bm), lambda i: (0, i))],
        out_specs=pl.BlockSpec((bm, d_out), lambda i: (i, 0)),
        out_shape=jax.ShapeDtypeStruct((npad, d_out), jnp.float32),
    )(xp, W, degp)

    partial = _make_edge_kernel(npad, d_out, NBLK, bs)(row, col, ew, h2)

    out = pl.pallas_call(
        _combine_body,
        grid=(npad // bm,),
        in_specs=[pl.BlockSpec((NC, bm, d_out), lambda i: (0, i, 0)),
                  pl.BlockSpec((NC, bm), lambda i: (0, i))],
        out_specs=pl.BlockSpec((bm, d_out), lambda i: (i, 0)),
        out_shape=jax.ShapeDtypeStruct((npad, d_out), jnp.float32),
    )(partial, degp)
    return out[:n]


# speedup vs baseline: 1.1127x; 1.1127x over previous
import jax
import jax.numpy as jnp
from jax import lax
from jax.experimental import pallas as pl
from jax.experimental.pallas import tpu as pltpu
from jax.experimental.pallas import tpu_sc as plsc

L = 16
NC = 2
NS = 16
NW = NC * NS
CH = 128
NBLK = 3

_SC_PARAMS = dict(
    compiler_params=pltpu.CompilerParams(needs_layout_passes=False),
)


def _make_deg_kernel(npad, nblk, bs):
    rpt = npad // NS

    def body(col3d, ew3d, degp_hbm, idx_c, ewb, zb, deg_sh, sem):
        c = lax.axis_index("c")
        s = lax.axis_index("s")
        wid = s * NC + c
        base_row = s * rpt

        zeros16 = jnp.zeros((L,), jnp.float32)
        for q in range(CH // L):
            zb[pl.ds(q * L, L)] = zeros16

        @pl.loop(0, rpt // CH)
        def _zd(k):
            pltpu.sync_copy(zb, deg_sh.at[pl.ds(base_row + k * CH, CH)])

        plsc.subcore_barrier()

        for b in range(nblk):
            pltpu.sync_copy(col3d.at[wid, b], idx_c)
            pltpu.sync_copy(ew3d.at[wid, b], ewb)

            @pl.loop(0, bs)
            def _fire(j):
                pltpu.async_copy(ewb.at[j], deg_sh.at[idx_c.at[j]], sem,
                                 add=True)

            @pl.loop(0, bs)
            def _drain(j):
                pltpu.make_async_copy(ewb.at[j], deg_sh.at[idx_c.at[j]],
                                      sem).wait()

        plsc.subcore_barrier()
        pltpu.sync_copy(deg_sh.at[pl.ds(base_row, rpt)],
                        degp_hbm.at[c, pl.ds(base_row, rpt)])

    mesh = plsc.VectorSubcoreMesh(core_axis_name="c", subcore_axis_name="s")
    return pl.kernel(
        body,
        out_type=jax.ShapeDtypeStruct((NC, npad), jnp.float32),
        mesh=mesh,
        scratch_types=[
            pltpu.VMEM((bs, CH), jnp.int32),
            pltpu.VMEM((bs, CH), jnp.float32),
            pltpu.VMEM((CH,), jnp.float32),
            pltpu.VMEM_SHARED((npad,), jnp.float32),
            pltpu.SemaphoreType.DMA,
        ],
        **_SC_PARAMS,
    )


def _make_edge_kernel(npad, d_out, nblk, bs):
    rpt = npad // NS
    qn = d_out // L

    def body(row3d, col3d, ew3d, h2, out_hbm,
             idx_r, idx_c, ewb, rows_a, rows_b, zbuf, acc_sh, sga, sgb):
        c = lax.axis_index("c")
        s = lax.axis_index("s")
        wid = s * NC + c
        base_row = s * rpt

        zeros16 = jnp.zeros((L,), jnp.float32)

        @pl.loop(0, 8)
        def _z(i):
            for q in range(qn):
                zbuf[i, pl.ds(q * L, L)] = zeros16

        @pl.loop(0, rpt // 8)
        def _za(k):
            pltpu.sync_copy(zbuf, acc_sh.at[pl.ds(base_row + k * 8, 8)])

        plsc.subcore_barrier()

        def proc(j, buf, sem):
            pltpu.make_async_copy(h2.at[idx_r.at[j]], buf, sem).wait()

            @pl.loop(0, CH // L)
            def _grp(g):
                goff = pl.multiple_of(g * L, L)
                ev = ewb[j, pl.ds(goff, L)]
                for i in range(L):
                    w = ev[i]
                    e_idx = goff + i
                    for q in range(qn):
                        buf[e_idx, pl.ds(q * L, L)] = (
                            buf[e_idx, pl.ds(q * L, L)] * w)

            pltpu.sync_copy(buf, acc_sh.at[idx_c.at[j]], add=True)

        for b in range(nblk):
            pltpu.sync_copy(row3d.at[wid, b], idx_r)
            pltpu.sync_copy(col3d.at[wid, b], idx_c)
            pltpu.sync_copy(ew3d.at[wid, b], ewb)
            pltpu.async_copy(h2.at[idx_r.at[0]], rows_a, sga)
            pltpu.async_copy(h2.at[idx_r.at[1]], rows_b, sgb)

            @pl.loop(0, bs // 2)
            def _pair(it):
                j0 = 2 * it
                proc(j0, rows_a, sga)

                @pl.when(j0 + 2 < bs)
                def _():
                    pltpu.async_copy(h2.at[idx_r.at[j0 + 2]], rows_a, sga)

                proc(j0 + 1, rows_b, sgb)

                @pl.when(j0 + 3 < bs)
                def _():
                    pltpu.async_copy(h2.at[idx_r.at[j0 + 3]], rows_b, sgb)

        plsc.subcore_barrier()
        pltpu.sync_copy(acc_sh.at[pl.ds(base_row, rpt)],
                        out_hbm.at[c, pl.ds(base_row, rpt)])

    mesh = plsc.VectorSubcoreMesh(core_axis_name="c", subcore_axis_name="s")
    return pl.kernel(
        body,
        out_type=jax.ShapeDtypeStruct((NC, npad, d_out), jnp.float32),
        mesh=mesh,
        scratch_types=[
            pltpu.VMEM((bs, CH), jnp.int32),
            pltpu.VMEM((bs, CH), jnp.int32),
            pltpu.VMEM((bs, CH), jnp.float32),
            pltpu.VMEM((CH, d_out), jnp.float32),
            pltpu.VMEM((CH, d_out), jnp.float32),
            pltpu.VMEM((8, d_out), jnp.float32),
            pltpu.VMEM_SHARED((npad, d_out), jnp.float32),
            pltpu.SemaphoreType.DMA,
            pltpu.SemaphoreType.DMA,
        ],
        **_SC_PARAMS,
    )


def _dis_block(degp_blk):
    deg = degp_blk[0] + degp_blk[1]
    return jnp.where(deg > 0.0, lax.rsqrt(jnp.where(deg > 0.0, deg, 1.0)),
                     0.0)


def _matmul_body(x_ref, w_ref, degp_ref, o_ref):
    dis = _dis_block(degp_ref[...])
    o_ref[...] = jnp.dot(x_ref[...], w_ref[...],
                         preferred_element_type=jnp.float32) * dis[:, None]


def _combine_body(p_ref, degp_ref, o_ref):
    dis = _dis_block(degp_ref[...])
    o_ref[...] = (p_ref[0] + p_ref[1]) * dis[:, None]


def kernel(x, edge_index, edge_weight, W):
    n, d_in = x.shape
    d_out = W.shape[1]
    e = edge_weight.shape[0]

    loop_idx = jnp.arange(n, dtype=edge_index.dtype)
    row = jnp.concatenate([edge_index[0], loop_idx])
    col = jnp.concatenate([edge_index[1], loop_idx])
    ew = jnp.concatenate([edge_weight, jnp.ones((n,), edge_weight.dtype)])
    e_tot = e + n
    grp = NW * CH
    cpw = (e_tot + grp - 1) // grp
    cpw = ((cpw + 2 * NBLK - 1) // (2 * NBLK)) * (2 * NBLK)
    e_pad = cpw * grp
    pad = e_pad - e_tot
    bs = cpw // NBLK
    shp = (NW, NBLK, bs, CH)
    row = jnp.concatenate([row, jnp.zeros((pad,), row.dtype)]).reshape(shp)
    col = jnp.concatenate([col, jnp.zeros((pad,), col.dtype)]).reshape(shp)
    ew = jnp.concatenate([ew, jnp.zeros((pad,), ew.dtype)]).reshape(shp)

    rpt = ((n + NS * CH - 1) // (NS * CH)) * CH
    npad = rpt * NS

    degp = _make_deg_kernel(npad, NBLK, bs)(col, ew)

    xp = jnp.concatenate(
        [x, jnp.zeros((npad - n, d_in), x.dtype)]) if npad > n else x
    bm = 1024
    h2 = pl.pallas_call(
        _matmul_body,
        grid=(npad // bm,),
        in_specs=[pl.BlockSpec((bm, d_in), lambda i: (i, 0)),
                  pl.BlockSpec((d_in, d_out), lambda i: (0, 0)),
                  pl.BlockSpec((NC, bm), lambda i: (0, i))],
        out_specs=pl.BlockSpec((bm, d_out), lambda i: (i, 0)),
        out_shape=jax.ShapeDtypeStruct((npad, d_out), jnp.float32),
    )(xp, W, degp)

    partial = _make_edge_kernel(npad, d_out, NBLK, bs)(row, col, ew, h2)

    out = pl.pallas_call(
        _combine_body,
        grid=(npad // bm,),
        in_specs=[pl.BlockSpec((NC, bm, d_out), lambda i: (0, i, 0)),
                  pl.BlockSpec((NC, bm), lambda i: (0, i))],
        out_specs=pl.BlockSpec((bm, d_out), lambda i: (i, 0)),
        out_shape=jax.ShapeDtypeStruct((npad, d_out), jnp.float32),
    )(partial, degp)
    return out[:n]


# speedup vs baseline: 2.2506x; 2.0227x over previous
import jax
import jax.numpy as jnp
import numpy as np
from jax import lax
from jax.experimental import pallas as pl
from jax.experimental.pallas import tpu as pltpu
from jax.experimental.pallas import tpu_sc as plsc

L = 16
NC = 2
NS = 16
NW = NC * NS
CH = 128

MAGIC = np.int32(0x5F3759DF)


def _rsqrt16(d):
    i = lax.bitcast_convert_type(d, jnp.int32)
    y = lax.bitcast_convert_type(MAGIC - (i >> 1), jnp.float32)
    hd = d * 0.5
    y = y * (1.5 - hd * y * y)
    y = y * (1.5 - hd * y * y)
    y = y * (1.5 - hd * y * y)
    return jnp.where(d > 0.0, y, 0.0)


def _make_sc_kernel(n, npad, d_out, nblk, bs):
    rpt = npad // NS
    qn = d_out // L

    def body(row3d, col3d, ew3d, h_hbm, out_hbm,
             dis_l, idx_r, idx_c, ewb, rows_v, zbuf, degbuf, disbuf,
             acc_sh, deg_sh, dis_sh, sem):
        c = lax.axis_index("c")
        s = lax.axis_index("s")
        wid = s * NC + c
        base_row = s * rpt

        zeros16 = jnp.zeros((L,), jnp.float32)

        @pl.loop(0, 32)
        def _z(i):
            for q in range(qn):
                zbuf[i, pl.ds(q * L, L)] = zeros16

        @pl.loop(0, rpt // 32)
        def _za(k):
            pltpu.sync_copy(zbuf, acc_sh.at[pl.ds(base_row + k * 32, 32)])

        @pl.loop(0, rpt // CH)
        def _zd(k):
            pltpu.sync_copy(zbuf.at[0], deg_sh.at[pl.ds(base_row + k * CH, CH)])

        plsc.subcore_barrier()

        for p in range(2):
            for b in range(nblk):
                pltpu.sync_copy(col3d.at[s * 2 + p, b], idx_c)
                pltpu.sync_copy(ew3d.at[s * 2 + p, b], ewb)

                @pl.loop(0, bs)
                def _fire(j):
                    pltpu.async_copy(ewb.at[j], deg_sh.at[idx_c.at[j]], sem,
                                     add=True)

                @pl.loop(0, bs)
                def _drain(j):
                    pltpu.make_async_copy(ewb.at[j], deg_sh.at[idx_c.at[j]],
                                          sem).wait()

        plsc.subcore_barrier()

        pltpu.sync_copy(deg_sh.at[pl.ds(base_row, rpt)], degbuf)

        @pl.loop(0, rpt // L)
        def _rs(k):
            disbuf[pl.ds(k * L, L)] = _rsqrt16(degbuf[pl.ds(k * L, L)])

        pltpu.sync_copy(disbuf, dis_sh.at[pl.ds(base_row, rpt)])
        plsc.subcore_barrier()
        pltpu.sync_copy(dis_sh, dis_l)

        for b in range(nblk):
            pltpu.sync_copy(row3d.at[wid, b], idx_r)
            pltpu.sync_copy(col3d.at[wid, b], idx_c)
            pltpu.sync_copy(ew3d.at[wid, b], ewb)

            @pl.loop(0, bs)
            def _edge(j):
                pltpu.async_copy(h_hbm.at[idx_r.at[j]], rows_v, sem).wait()
                for g in range(CH // L):
                    rv = idx_r[j, pl.ds(g * L, L)]
                    cv = idx_c[j, pl.ds(g * L, L)]
                    ev = ewb[j, pl.ds(g * L, L)]
                    dr = plsc.load_gather(dis_l, [rv])
                    dc = plsc.load_gather(dis_l, [cv])
                    nv = dr * ev * dc
                    for i in range(L):
                        w = nv[i]
                        e_idx = g * L + i
                        for q in range(qn):
                            rows_v[e_idx, pl.ds(q * L, L)] = (
                                rows_v[e_idx, pl.ds(q * L, L)] * w)

                pltpu.sync_copy(rows_v, acc_sh.at[idx_c.at[j]], add=True)

        plsc.subcore_barrier()

        pltpu.sync_copy(acc_sh.at[pl.ds(base_row, rpt)],
                        out_hbm.at[c, pl.ds(base_row, rpt)])

    mesh = plsc.VectorSubcoreMesh(core_axis_name="c", subcore_axis_name="s")
    return pl.kernel(
        body,
        out_type=jax.ShapeDtypeStruct((NC, npad, d_out), jnp.float32),
        mesh=mesh,
        compiler_params=pltpu.CompilerParams(needs_layout_passes=False),
        scratch_types=[
            pltpu.VMEM((npad,), jnp.float32),
            pltpu.VMEM((bs, CH), jnp.int32),
            pltpu.VMEM((bs, CH), jnp.int32),
            pltpu.VMEM((bs, CH), jnp.float32),
            pltpu.VMEM((CH, d_out), jnp.float32),
            pltpu.VMEM((32, d_out), jnp.float32),
            pltpu.VMEM((rpt,), jnp.float32),
            pltpu.VMEM((rpt,), jnp.float32),
            pltpu.VMEM_SHARED((npad, d_out), jnp.float32),
            pltpu.VMEM_SHARED((npad,), jnp.float32),
            pltpu.VMEM_SHARED((npad,), jnp.float32),
            pltpu.SemaphoreType.DMA,
        ],
    )


def _matmul_body(x_ref, w_ref, o_ref):
    o_ref[...] = jnp.dot(x_ref[...], w_ref[...],
                         preferred_element_type=jnp.float32)


def _combine_body(p_ref, o_ref):
    o_ref[...] = p_ref[0] + p_ref[1]


def kernel(x, edge_index, edge_weight, W):
    n, d_in = x.shape
    d_out = W.shape[1]
    e = edge_weight.shape[0]

    loop_idx = jnp.arange(n, dtype=edge_index.dtype)
    row = jnp.concatenate([edge_index[0], loop_idx])
    col = jnp.concatenate([edge_index[1], loop_idx])
    ew = jnp.concatenate([edge_weight, jnp.ones((n,), edge_weight.dtype)])
    e_tot = e + n
    grp = NW * CH
    e_pad = ((e_tot + grp - 1) // grp) * grp
    pad = e_pad - e_tot
    cpw = e_pad // CH // NW
    nblk = 3 if cpw % 3 == 0 else 1
    shp = (NW, nblk, cpw // nblk, CH)
    row = jnp.concatenate([row, jnp.zeros((pad,), row.dtype)]).reshape(shp)
    col = jnp.concatenate([col, jnp.zeros((pad,), col.dtype)]).reshape(shp)
    ew = jnp.concatenate([ew, jnp.zeros((pad,), ew.dtype)]).reshape(shp)

    rpt = ((n + NS * 32 - 1) // (NS * 32)) * 32
    npad = rpt * NS

    bm = 1000 if n % 1000 == 0 else (625 if n % 625 == 0 else n)
    h = pl.pallas_call(
        _matmul_body,
        grid=(n // bm,),
        in_specs=[pl.BlockSpec((bm, d_in), lambda i: (i, 0)),
                  pl.BlockSpec((d_in, d_out), lambda i: (0, 0))],
        out_specs=pl.BlockSpec((bm, d_out), lambda i: (i, 0)),
        out_shape=jax.ShapeDtypeStruct((n, d_out), jnp.float32),
    )(x, W)

    partial = _make_sc_kernel(n, npad, d_out, nblk, cpw // nblk)(
        row, col, ew, h)

    out = pl.pallas_call(
        _combine_body,
        grid=(n // bm,),
        in_specs=[pl.BlockSpec((NC, bm, d_out), lambda i: (0, i, 0))],
        out_specs=pl.BlockSpec((bm, d_out), lambda i: (i, 0)),
        out_shape=jax.ShapeDtypeStruct((n, d_out), jnp.float32),
    )(partial)
    return out


# speedup vs baseline: 2.5993x; 1.1549x over previous
import jax
import jax.numpy as jnp
import numpy as np
from jax import lax
from jax.experimental import pallas as pl
from jax.experimental.pallas import tpu as pltpu
from jax.experimental.pallas import tpu_sc as plsc

L = 16
NC = 2
NS = 16
NW = NC * NS
CH = 128

MAGIC = np.int32(0x5F3759DF)


def _rsqrt16(d):
    i = lax.bitcast_convert_type(d, jnp.int32)
    y = lax.bitcast_convert_type(MAGIC - (i >> 1), jnp.float32)
    hd = d * 0.5
    y = y * (1.5 - hd * y * y)
    y = y * (1.5 - hd * y * y)
    y = y * (1.5 - hd * y * y)
    return jnp.where(d > 0.0, y, 0.0)


def _make_sc_kernel(npad, d_out, nblk, bs):
    rpt = npad // NS
    qn = d_out // L

    def body(row3d, col3d, ew3d, h_hbm, out_hbm, h2_hbm, dis_hbm,
             idx_r, idx_c, ewb, rows_a, rows_b, zbuf, degbuf,
             acc_sh, deg_sh, sga, sgb):
        c = lax.axis_index("c")
        s = lax.axis_index("s")
        wid = s * NC + c
        base_row = s * rpt

        zeros16 = jnp.zeros((L,), jnp.float32)

        @pl.loop(0, 8)
        def _z(i):
            for q in range(qn):
                zbuf[i, pl.ds(q * L, L)] = zeros16

        @pl.loop(0, rpt // 8)
        def _za(k):
            pltpu.sync_copy(zbuf, acc_sh.at[pl.ds(base_row + k * 8, 8)])

        @pl.loop(0, rpt // CH)
        def _zd(k):
            pltpu.sync_copy(zbuf.at[0], deg_sh.at[pl.ds(base_row + k * CH, CH)])

        plsc.subcore_barrier()

        for p in range(2):
            for b in range(nblk):
                pltpu.sync_copy(col3d.at[s * 2 + p, b], idx_c)
                pltpu.sync_copy(ew3d.at[s * 2 + p, b], ewb)

                @pl.loop(0, bs)
                def _fire(j):
                    pltpu.async_copy(ewb.at[j], deg_sh.at[idx_c.at[j]], sga,
                                     add=True)

                @pl.loop(0, bs)
                def _drain(j):
                    pltpu.make_async_copy(ewb.at[j], deg_sh.at[idx_c.at[j]],
                                          sga).wait()

        plsc.subcore_barrier()

        pltpu.sync_copy(deg_sh.at[pl.ds(base_row, rpt)], degbuf)

        @pl.loop(0, rpt // L)
        def _rs(k):
            koff = pl.multiple_of(k * L, L)
            degbuf[pl.ds(koff, L)] = _rsqrt16(degbuf[pl.ds(koff, L)])

        pltpu.sync_copy(degbuf, dis_hbm.at[c, pl.ds(base_row, rpt)])

        @pl.loop(0, rpt // CH)
        def _h2(k):
            roff = pl.multiple_of(k * CH, CH)
            pltpu.sync_copy(h_hbm.at[pl.ds(base_row + roff, CH)], rows_a)

            @pl.loop(0, CH // L)
            def _hg(g):
                goff = pl.multiple_of(g * L, L)
                dv = degbuf[pl.ds(roff + goff, L)]
                for i in range(L):
                    w = dv[i]
                    r_idx = goff + i
                    for q in range(qn):
                        rows_a[r_idx, pl.ds(q * L, L)] = (
                            rows_a[r_idx, pl.ds(q * L, L)] * w)

            pltpu.sync_copy(
                rows_a, h2_hbm.at[pl.ds(c * npad + base_row + roff, CH)])

        plsc.subcore_barrier()

        coff_c = c * npad

        def proc(j, buf, sem):
            pltpu.make_async_copy(h2_hbm.at[idx_r.at[j]], buf, sem).wait()

            @pl.loop(0, CH // L)
            def _grp(g):
                goff = pl.multiple_of(g * L, L)
                ev = ewb[j, pl.ds(goff, L)]
                for i in range(L):
                    w = ev[i]
                    e_idx = goff + i
                    for q in range(qn):
                        buf[e_idx, pl.ds(q * L, L)] = (
                            buf[e_idx, pl.ds(q * L, L)] * w)

            pltpu.sync_copy(buf, acc_sh.at[idx_c.at[j]], add=True)

        def issue(j, buf, sem):
            pltpu.async_copy(h2_hbm.at[idx_r.at[j]], buf, sem)

        for b in range(nblk):
            pltpu.sync_copy(row3d.at[wid, b], idx_r)
            pltpu.sync_copy(col3d.at[wid, b], idx_c)
            pltpu.sync_copy(ew3d.at[wid, b], ewb)

            @pl.loop(0, bs)
            def _rb(j):
                for g in range(CH // L):
                    idx_r[j, pl.ds(g * L, L)] = (
                        idx_r[j, pl.ds(g * L, L)] + coff_c)

            issue(0, rows_a, sga)
            issue(1, rows_b, sgb)

            @pl.loop(0, bs // 2)
            def _pair(it):
                j0 = 2 * it
                proc(j0, rows_a, sga)

                @pl.when(j0 + 2 < bs)
                def _():
                    issue(j0 + 2, rows_a, sga)

                proc(j0 + 1, rows_b, sgb)

                @pl.when(j0 + 3 < bs)
                def _():
                    issue(j0 + 3, rows_b, sgb)

            if bs % 2 == 1:
                proc(bs - 1, rows_a, sga)

        plsc.subcore_barrier()

        pltpu.sync_copy(acc_sh.at[pl.ds(base_row, rpt)],
                        out_hbm.at[c, pl.ds(base_row, rpt)])

    mesh = plsc.VectorSubcoreMesh(core_axis_name="c", subcore_axis_name="s")
    return pl.kernel(
        body,
        out_type=(
            jax.ShapeDtypeStruct((NC, npad, d_out), jnp.float32),
            jax.ShapeDtypeStruct((NC * npad, d_out), jnp.float32),
            jax.ShapeDtypeStruct((NC, npad), jnp.float32),
        ),
        mesh=mesh,
        compiler_params=pltpu.CompilerParams(needs_layout_passes=False),
        scratch_types=[
            pltpu.VMEM((bs, CH), jnp.int32),
            pltpu.VMEM((bs, CH), jnp.int32),
            pltpu.VMEM((bs, CH), jnp.float32),
            pltpu.VMEM((CH, d_out), jnp.float32),
            pltpu.VMEM((CH, d_out), jnp.float32),
            pltpu.VMEM((8, d_out), jnp.float32),
            pltpu.VMEM((rpt,), jnp.float32),
            pltpu.VMEM_SHARED((npad, d_out), jnp.float32),
            pltpu.VMEM_SHARED((npad,), jnp.float32),
            pltpu.SemaphoreType.DMA,
            pltpu.SemaphoreType.DMA,
        ],
    )


def _matmul_body(x_ref, w_ref, o_ref):
    o_ref[...] = jnp.dot(x_ref[...], w_ref[...],
                         preferred_element_type=jnp.float32)


def _combine_body(p_ref, d_ref, o_ref):
    o_ref[...] = (p_ref[0] + p_ref[1]) * d_ref[0][:, None]


def kernel(x, edge_index, edge_weight, W):
    n, d_in = x.shape
    d_out = W.shape[1]
    e = edge_weight.shape[0]

    loop_idx = jnp.arange(n, dtype=edge_index.dtype)
    row = jnp.concatenate([edge_index[0], loop_idx])
    col = jnp.concatenate([edge_index[1], loop_idx])
    ew = jnp.concatenate([edge_weight, jnp.ones((n,), edge_weight.dtype)])
    e_tot = e + n
    grp = NW * CH
    e_pad = ((e_tot + grp - 1) // grp) * grp
    pad = e_pad - e_tot
    cpw = e_pad // CH // NW
    nblk = 3 if cpw % 3 == 0 else 1
    shp = (NW, nblk, cpw // nblk, CH)
    row = jnp.concatenate([row, jnp.zeros((pad,), row.dtype)]).reshape(shp)
    col = jnp.concatenate([col, jnp.zeros((pad,), col.dtype)]).reshape(shp)
    ew = jnp.concatenate([ew, jnp.zeros((pad,), ew.dtype)]).reshape(shp)

    rpt = ((n + NS * CH - 1) // (NS * CH)) * CH
    npad = rpt * NS

    xp = jnp.concatenate(
        [x, jnp.zeros((npad - n, d_in), x.dtype)]) if npad > n else x
    bm = 1024
    h = pl.pallas_call(
        _matmul_body,
        grid=(npad // bm,),
        in_specs=[pl.BlockSpec((bm, d_in), lambda i: (i, 0)),
                  pl.BlockSpec((d_in, d_out), lambda i: (0, 0))],
        out_specs=pl.BlockSpec((bm, d_out), lambda i: (i, 0)),
        out_shape=jax.ShapeDtypeStruct((npad, d_out), jnp.float32),
    )(xp, W)

    partial, _h2, dis = _make_sc_kernel(npad, d_out, nblk, cpw // nblk)(
        row, col, ew, h)

    out = pl.pallas_call(
        _combine_body,
        grid=(npad // bm,),
        in_specs=[pl.BlockSpec((NC, bm, d_out), lambda i: (0, i, 0)),
                  pl.BlockSpec((NC, bm), lambda i: (0, i))],
        out_specs=pl.BlockSpec((bm, d_out), lambda i: (i, 0)),
        out_shape=jax.ShapeDtypeStruct((npad, d_out), jnp.float32),
    )(partial, dis)
    return out[:n]
